# Initial kernel scaffold; baseline (speedup 1.0000x reference)
import functools

import jax
import jax.numpy as jnp
from jax import lax
from jax.experimental import pallas as pl
from jax.experimental.pallas import tpu as pltpu
from jax.experimental.pallas import tpu_sc as plsc

N = 50000
E = 800000
H = 4
C = 32
HC = 128

NC = 2
NS = 16
NW = NC * NS
W_PER = 25600
EP = NW * W_PER
CH = 1024
CHUNKS = W_PER // CH
GROUPS = CH // 16
ZR = N // NS

_mesh = plsc.VectorSubcoreMesh(
    core_axis_name="c", subcore_axis_name="s", num_cores=NC, num_subcores=NS)


@functools.partial(
    pl.kernel,
    out_type=jax.ShapeDtypeStruct((4, EP), jnp.float32),
    mesh=_mesh,
    scratch_types=[
        pltpu.VMEM((2 * N,), jnp.float32),
        pltpu.VMEM((CH,), jnp.int32),
        pltpu.VMEM((CH,), jnp.int32),
        pltpu.VMEM((4, CH), jnp.float32),
    ],
)
def _sc_gather(x_hbm, src_hbm, dst_hbm, out_hbm, xtab, sbuf, dbuf, cbuf):
    c = lax.axis_index("c")
    s = lax.axis_index("s")
    wid = s * NC + c
    pltpu.sync_copy(x_hbm, xtab)
    base = wid * W_PER

    @pl.loop(0, CHUNKS)
    def _chunk(ci):
        off = base + ci * CH
        pltpu.sync_copy(src_hbm.at[pl.ds(off, CH)], sbuf)
        pltpu.sync_copy(dst_hbm.at[pl.ds(off, CH)], dbuf)

        @pl.loop(0, GROUPS)
        def _grp(g):
            si = sbuf[pl.ds(g * 16, 16)] * 2
            di = dbuf[pl.ds(g * 16, 16)] * 2
            cbuf[0, pl.ds(g * 16, 16)] = plsc.load_gather(xtab, [si])
            cbuf[1, pl.ds(g * 16, 16)] = plsc.load_gather(xtab, [si + 1])
            cbuf[2, pl.ds(g * 16, 16)] = plsc.load_gather(xtab, [di])
            cbuf[3, pl.ds(g * 16, 16)] = plsc.load_gather(xtab, [di + 1])

        pltpu.sync_copy(cbuf, out_hbm.at[:, pl.ds(off, CH)])


@functools.partial(
    pl.kernel,
    out_type=jax.ShapeDtypeStruct((NC, N, 16), jnp.float32),
    mesh=_mesh,
    scratch_types=[
        pltpu.VMEM_SHARED((N, 16), jnp.float32),
        pltpu.VMEM((8, 128), jnp.int32),
        pltpu.VMEM((4, CH), jnp.float32),
        pltpu.VMEM((2, CH), jnp.float32),
        pltpu.VMEM((CH, 16), jnp.float32),
        pltpu.VMEM((ZR, 16), jnp.float32),
    ],
)
def _sc_scatter(ex_hbm, co_hbm, dst_hbm, out_hbm,
                tsh, dbuf, exbuf, abbuf, vbuf, zbuf):
    c = lax.axis_index("c")
    s = lax.axis_index("s")
    wid = s * NC + c

    @pl.loop(0, ZR)
    def _z(i):
        zbuf[i, :] = jnp.zeros((16,), jnp.float32)

    pltpu.sync_copy(zbuf.at[pl.ds(0, CH)], vbuf)
    pltpu.sync_copy(zbuf, tsh.at[pl.ds(s * ZR, ZR)])
    plsc.subcore_barrier()

    base = wid * W_PER

    @pl.loop(0, CHUNKS)
    def _chunk(ci):
        off = base + ci * CH
        pltpu.sync_copy(ex_hbm.at[:, pl.ds(off, CH)], exbuf)
        pltpu.sync_copy(co_hbm.at[pl.ds(0, 2), pl.ds(off, CH)], abbuf)
        pltpu.sync_copy(dst_hbm.at[pl.ds(off // 128, 8)], dbuf)

        @pl.loop(0, GROUPS)
        def _grp(g):
            rows = g * 16 + lax.iota(jnp.int32, 16)
            a = abbuf[0, pl.ds(g * 16, 16)]
            b = abbuf[1, pl.ds(g * 16, 16)]
            for h in range(H):
                exh = exbuf[h, pl.ds(g * 16, 16)]
                col = jnp.full((16,), h, jnp.int32)
                plsc.store_scatter(vbuf, [rows, col], exh)
                plsc.store_scatter(vbuf, [rows, col + 4], exh * a)
                plsc.store_scatter(vbuf, [rows, col + 8], exh * b)

        for j in range(8):
            pltpu.sync_copy(vbuf.at[pl.ds(j * 128, 128)],
                            tsh.at[dbuf.at[j]], add=True)

    plsc.subcore_barrier()
    pltpu.sync_copy(tsh.at[pl.ds(s * ZR, ZR)],
                    out_hbm.at[c, pl.ds(s * ZR, ZR)])


EB = 2048


def _tc_dense_body(co_ref, ea_ref, w5t_ref, attc_ref, out_ref):
    coef5 = jnp.concatenate([co_ref[...], ea_ref[...]], axis=0)
    m = lax.dot_general(w5t_ref[...], coef5, (((1,), (0,)), ((), ())),
                        preferred_element_type=jnp.float32,
                        precision=lax.Precision.HIGHEST)
    g = jnp.where(m >= 0.0, m, 0.2 * m) * attc_ref[...]
    hs = (lax.broadcasted_iota(jnp.int32, (H, HC), 1) // C
          == lax.broadcasted_iota(jnp.int32, (H, HC), 0)).astype(jnp.float32)
    logits = lax.dot_general(hs, g, (((1,), (0,)), ((), ())),
                             preferred_element_type=jnp.float32,
                             precision=lax.Precision.HIGHEST)
    eidx = pl.program_id(0) * EB + lax.broadcasted_iota(jnp.int32, (H, EB), 1)
    out_ref[...] = jnp.where(eidx < E, jnp.exp(logits), 0.0)


_tc_dense = pl.pallas_call(
    _tc_dense_body,
    grid=(EP // EB,),
    in_specs=[
        pl.BlockSpec((4, EB), lambda i: (0, i)),
        pl.BlockSpec((1, EB), lambda i: (0, i)),
        pl.BlockSpec((HC, 5), lambda i: (0, 0)),
        pl.BlockSpec((HC, 1), lambda i: (0, 0)),
    ],
    out_specs=pl.BlockSpec((4, EB), lambda i: (0, i)),
    out_shape=jax.ShapeDtypeStruct((4, EP), jnp.float32),
)


NB = 1000


def _tc_stats_body(t_ref, out_ref):
    i = pl.program_id(0)
    ts = t_ref[0] + t_ref[1]
    den = ts[:, 0:4] + 1e-16
    sa = ts[:, 4:8] / den
    sb = ts[:, 8:12] / den
    m1 = jnp.sum(jnp.concatenate([sa, sb], axis=1), axis=0, keepdims=True)
    m2 = jnp.sum(jnp.concatenate([sa * sa, sb * sb], axis=1),
                 axis=0, keepdims=True)
    m3 = jnp.sum(jnp.concatenate([sa * sb, sa * sb], axis=1),
                 axis=0, keepdims=True)
    blk = jnp.concatenate([m1, m2, m3], axis=0)
    blk = jnp.concatenate([blk, jnp.zeros((3, 120), jnp.float32)], axis=1)
    blk = jnp.concatenate([blk, jnp.zeros((5, 128), jnp.float32)], axis=0)

    @pl.when(i == 0)
    def _():
        out_ref[...] = jnp.zeros_like(out_ref)

    out_ref[...] += blk


_tc_stats = pl.pallas_call(
    _tc_stats_body,
    grid=(N // NB,),
    in_specs=[pl.BlockSpec((NC, NB, 16), lambda i: (0, i, 0))],
    out_specs=pl.BlockSpec((8, 128), lambda i: (0, 0)),
    out_shape=jax.ShapeDtypeStruct((8, 128), jnp.float32),
)


def _tc_out_body(t_ref, st_ref, wl_ref, gam_ref, bet_ref, y_ref):
    ts = t_ref[0] + t_ref[1]
    den = ts[:, 0:4] + 1e-16
    sa = ts[:, 4:8] / den
    sb = ts[:, 8:12] / den
    st = st_ref[...]
    inv_n = 1.0 / N
    m1 = st[0:1, 0:8] * inv_n
    m2 = st[1:2, 0:8] * inv_n
    m3 = st[2:3, 0:8] * inv_n
    am, bm = m1[:, 0:4], m1[:, 4:8]
    var_a = m2[:, 0:4] - am * am
    var_b = m2[:, 4:8] - bm * bm
    cov = m3[:, 0:4] - am * bm
    hs = (lax.broadcasted_iota(jnp.int32, (H, HC), 1) // C
          == lax.broadcasted_iota(jnp.int32, (H, HC), 0)).astype(jnp.float32)

    def expand(z):
        return lax.dot_general(z, hs, (((1,), (0,)), ((), ())),
                               preferred_element_type=jnp.float32,
                               precision=lax.Precision.HIGHEST)

    u = wl_ref[0:1, :]
    v = wl_ref[1:2, :]
    var = (u * u * expand(var_a) + v * v * expand(var_b)
           + 2.0 * u * v * expand(cov))
    sig = jnp.sqrt(var + 1e-5)
    p = gam_ref[...] * u / sig
    q = gam_ref[...] * v / sig
    y_ref[...] = ((expand(sa) - expand(am)) * p
                  + (expand(sb) - expand(bm)) * q + bet_ref[...])


_tc_out = pl.pallas_call(
    _tc_out_body,
    grid=(N // NB,),
    in_specs=[
        pl.BlockSpec((NC, NB, 16), lambda i: (0, i, 0)),
        pl.BlockSpec((8, 128), lambda i: (0, 0)),
        pl.BlockSpec((2, HC), lambda i: (0, 0)),
        pl.BlockSpec((1, HC), lambda i: (0, 0)),
        pl.BlockSpec((1, HC), lambda i: (0, 0)),
    ],
    out_specs=pl.BlockSpec((NB, HC), lambda i: (i, 0)),
    out_shape=jax.ShapeDtypeStruct((N, HC), jnp.float32),
)


def kernel(x, edge_index, edge_attr, Wl, Wr, We, att, bias, gamma, beta):
    del bias
    pad = EP - E
    src = jnp.concatenate([edge_index[0], jnp.zeros((pad,), jnp.int32)])
    dst = jnp.concatenate([edge_index[1], jnp.zeros((pad,), jnp.int32)])
    dst_r = dst.reshape(EP // 128, 128)
    ea = jnp.concatenate([edge_attr[:, 0],
                          jnp.zeros((pad,), jnp.float32)]).reshape(1, EP)
    w5t = jnp.stack([Wl[0], Wl[1], Wr[0], Wr[1], We[0]], axis=1)
    att_col = att.reshape(HC, 1)

    co = _sc_gather(x.reshape(-1), src, dst)
    ex = _tc_dense(co, ea, w5t, att_col)
    t = _sc_scatter(ex, co, dst_r)
    st = _tc_stats(t)
    return _tc_out(t, st, Wl, gamma.reshape(1, HC), beta.reshape(1, HC))


# speedup vs baseline: 93.3207x; 93.3207x over previous
import functools

import jax
import jax.numpy as jnp
from jax import lax
from jax.experimental import pallas as pl
from jax.experimental.pallas import tpu as pltpu
from jax.experimental.pallas import tpu_sc as plsc

N = 50000
E = 800000
H = 4
C = 32
HC = 128

NC = 2
NS = 16
NW = NC * NS
W_PER = 25600
EP = NW * W_PER
CH = 1024
CHUNKS = W_PER // CH
GROUPS = CH // 16
NP = 50048
ZR = NP // NS

_mesh = plsc.VectorSubcoreMesh(
    core_axis_name="c", subcore_axis_name="s", num_cores=NC, num_subcores=NS)
_sc_params = pltpu.CompilerParams(
    needs_layout_passes=False, use_tc_tiling_on_sc=False)


@functools.partial(
    pl.kernel,
    out_type=jax.ShapeDtypeStruct((4, EP), jnp.float32),
    mesh=_mesh,
    scratch_types=[
        pltpu.VMEM((2 * N,), jnp.float32),
        pltpu.VMEM((CH,), jnp.int32),
        pltpu.VMEM((CH,), jnp.int32),
        pltpu.VMEM((4, CH), jnp.float32),
    ],
    compiler_params=_sc_params,
)
def _sc_gather(x_hbm, src_hbm, dst_hbm, out_hbm, xtab, sbuf, dbuf, cbuf):
    c = lax.axis_index("c")
    s = lax.axis_index("s")
    wid = s * NC + c
    pltpu.sync_copy(x_hbm, xtab)
    base = wid * W_PER

    @pl.loop(0, CHUNKS)
    def _chunk(ci):
        off = pl.multiple_of(base + ci * CH, CH)
        pltpu.sync_copy(src_hbm.at[pl.ds(off, CH)], sbuf)
        pltpu.sync_copy(dst_hbm.at[pl.ds(off, CH)], dbuf)

        @pl.loop(0, GROUPS)
        def _grp(g):
            si = sbuf[pl.ds(g * 16, 16)] * 2
            di = dbuf[pl.ds(g * 16, 16)] * 2
            cbuf[0, pl.ds(g * 16, 16)] = plsc.load_gather(xtab, [si])
            cbuf[1, pl.ds(g * 16, 16)] = plsc.load_gather(xtab, [si + 1])
            cbuf[2, pl.ds(g * 16, 16)] = plsc.load_gather(xtab, [di])
            cbuf[3, pl.ds(g * 16, 16)] = plsc.load_gather(xtab, [di + 1])

        pltpu.sync_copy(cbuf, out_hbm.at[:, pl.ds(off, CH)])


@functools.partial(
    pl.kernel,
    out_type=jax.ShapeDtypeStruct((NC, NP, 16), jnp.float32),
    mesh=_mesh,
    scratch_types=[
        pltpu.VMEM_SHARED((NP, 16), jnp.float32),
        pltpu.VMEM((8, 128), jnp.int32),
        pltpu.VMEM((4, CH), jnp.float32),
        pltpu.VMEM((2, CH), jnp.float32),
        pltpu.VMEM((CH, 16), jnp.float32),
        pltpu.VMEM((ZR, 16), jnp.float32),
    ],
    compiler_params=_sc_params,
)
def _sc_scatter(ex_hbm, co_hbm, dst_hbm, out_hbm,
                tsh, dbuf, exbuf, abbuf, vbuf, zbuf):
    c = lax.axis_index("c")
    s = lax.axis_index("s")
    wid = s * NC + c

    @pl.loop(0, ZR)
    def _z(i):
        zbuf[i, :] = jnp.zeros((16,), jnp.float32)

    @pl.loop(0, CH)
    def _zv(i):
        vbuf[i, :] = jnp.zeros((16,), jnp.float32)

    pltpu.sync_copy(zbuf, tsh.at[pl.ds(s * ZR, ZR)])
    plsc.subcore_barrier()

    base = wid * W_PER

    @pl.loop(0, CHUNKS)
    def _chunk(ci):
        off = pl.multiple_of(base + ci * CH, CH)
        pltpu.sync_copy(ex_hbm.at[:, pl.ds(off, CH)], exbuf)
        pltpu.sync_copy(co_hbm.at[pl.ds(0, 2), pl.ds(off, CH)], abbuf)
        pltpu.sync_copy(dst_hbm.at[pl.ds(pl.multiple_of(off // 128, 8), 8)],
                        dbuf)

        @pl.loop(0, GROUPS)
        def _grp(g):
            rows = g * 16 + lax.iota(jnp.int32, 16)
            a = abbuf[0, pl.ds(g * 16, 16)]
            b = abbuf[1, pl.ds(g * 16, 16)]
            for h in range(H):
                exh = exbuf[h, pl.ds(g * 16, 16)]
                col = jnp.full((16,), h, jnp.int32)
                plsc.store_scatter(vbuf, [rows, col], exh)
                plsc.store_scatter(vbuf, [rows, col + 4], exh * a)
                plsc.store_scatter(vbuf, [rows, col + 8], exh * b)

        for j in range(8):
            pltpu.sync_copy(vbuf.at[pl.ds(j * 128, 128)],
                            tsh.at[dbuf.at[j]], add=True)

    plsc.subcore_barrier()
    pltpu.sync_copy(tsh.at[pl.ds(s * ZR, ZR)],
                    out_hbm.at[c, pl.ds(s * ZR, ZR)])


EB = 2048


def _tc_dense_body(co_ref, ea_ref, w5t_ref, attc_ref, out_ref):
    coef5 = jnp.concatenate([co_ref[...], ea_ref[...]], axis=0)
    m = lax.dot_general(w5t_ref[...], coef5, (((1,), (0,)), ((), ())),
                        preferred_element_type=jnp.float32,
                        precision=lax.Precision.HIGHEST)
    g = jnp.where(m >= 0.0, m, 0.2 * m) * attc_ref[...]
    hs = (lax.broadcasted_iota(jnp.int32, (H, HC), 1) // C
          == lax.broadcasted_iota(jnp.int32, (H, HC), 0)).astype(jnp.float32)
    logits = lax.dot_general(hs, g, (((1,), (0,)), ((), ())),
                             preferred_element_type=jnp.float32,
                             precision=lax.Precision.HIGHEST)
    eidx = pl.program_id(0) * EB + lax.broadcasted_iota(jnp.int32, (H, EB), 1)
    out_ref[...] = jnp.where(eidx < E, jnp.exp(logits), 0.0)


_tc_dense = pl.pallas_call(
    _tc_dense_body,
    grid=(EP // EB,),
    in_specs=[
        pl.BlockSpec((4, EB), lambda i: (0, i)),
        pl.BlockSpec((1, EB), lambda i: (0, i)),
        pl.BlockSpec((HC, 5), lambda i: (0, 0)),
        pl.BlockSpec((HC, 1), lambda i: (0, 0)),
    ],
    out_specs=pl.BlockSpec((4, EB), lambda i: (0, i)),
    out_shape=jax.ShapeDtypeStruct((4, EP), jnp.float32),
)


NB = 1000


def _tc_stats_body(t_ref, out_ref):
    i = pl.program_id(0)
    ts = t_ref[0] + t_ref[1]
    den = ts[:, 0:4] + 1e-16
    sa = ts[:, 4:8] / den
    sb = ts[:, 8:12] / den
    m1 = jnp.sum(jnp.concatenate([sa, sb], axis=1), axis=0, keepdims=True)
    m2 = jnp.sum(jnp.concatenate([sa * sa, sb * sb], axis=1),
                 axis=0, keepdims=True)
    m3 = jnp.sum(jnp.concatenate([sa * sb, sa * sb], axis=1),
                 axis=0, keepdims=True)
    blk = jnp.concatenate([m1, m2, m3], axis=0)
    blk = jnp.concatenate([blk, jnp.zeros((3, 120), jnp.float32)], axis=1)
    blk = jnp.concatenate([blk, jnp.zeros((5, 128), jnp.float32)], axis=0)

    @pl.when(i == 0)
    def _():
        out_ref[...] = jnp.zeros_like(out_ref)

    out_ref[...] += blk


_tc_stats = pl.pallas_call(
    _tc_stats_body,
    grid=(N // NB,),
    in_specs=[pl.BlockSpec((NC, NB, 16), lambda i: (0, i, 0))],
    out_specs=pl.BlockSpec((8, 128), lambda i: (0, 0)),
    out_shape=jax.ShapeDtypeStruct((8, 128), jnp.float32),
)


def _tc_out_body(t_ref, st_ref, wl_ref, gam_ref, bet_ref, y_ref):
    ts = t_ref[0] + t_ref[1]
    den = ts[:, 0:4] + 1e-16
    sa = ts[:, 4:8] / den
    sb = ts[:, 8:12] / den
    st = st_ref[...]
    inv_n = 1.0 / N
    m1 = st[0:1, 0:8] * inv_n
    m2 = st[1:2, 0:8] * inv_n
    m3 = st[2:3, 0:8] * inv_n
    am, bm = m1[:, 0:4], m1[:, 4:8]
    var_a = m2[:, 0:4] - am * am
    var_b = m2[:, 4:8] - bm * bm
    cov = m3[:, 0:4] - am * bm
    hs = (lax.broadcasted_iota(jnp.int32, (H, HC), 1) // C
          == lax.broadcasted_iota(jnp.int32, (H, HC), 0)).astype(jnp.float32)

    def expand(z):
        return lax.dot_general(z, hs, (((1,), (0,)), ((), ())),
                               preferred_element_type=jnp.float32,
                               precision=lax.Precision.HIGHEST)

    u = wl_ref[0:1, :]
    v = wl_ref[1:2, :]
    var = (u * u * expand(var_a) + v * v * expand(var_b)
           + 2.0 * u * v * expand(cov))
    sig = jnp.sqrt(var + 1e-5)
    p = gam_ref[...] * u / sig
    q = gam_ref[...] * v / sig
    y_ref[...] = ((expand(sa) - expand(am)) * p
                  + (expand(sb) - expand(bm)) * q + bet_ref[...])


_tc_out = pl.pallas_call(
    _tc_out_body,
    grid=(N // NB,),
    in_specs=[
        pl.BlockSpec((NC, NB, 16), lambda i: (0, i, 0)),
        pl.BlockSpec((8, 128), lambda i: (0, 0)),
        pl.BlockSpec((2, HC), lambda i: (0, 0)),
        pl.BlockSpec((1, HC), lambda i: (0, 0)),
        pl.BlockSpec((1, HC), lambda i: (0, 0)),
    ],
    out_specs=pl.BlockSpec((NB, HC), lambda i: (i, 0)),
    out_shape=jax.ShapeDtypeStruct((N, HC), jnp.float32),
)


def kernel(x, edge_index, edge_attr, Wl, Wr, We, att, bias, gamma, beta):
    del bias
    pad = EP - E
    src = jnp.concatenate([edge_index[0], jnp.zeros((pad,), jnp.int32)])
    dst = jnp.concatenate([edge_index[1], jnp.zeros((pad,), jnp.int32)])
    dst_r = dst.reshape(EP // 128, 128)
    ea = jnp.concatenate([edge_attr[:, 0],
                          jnp.zeros((pad,), jnp.float32)]).reshape(1, EP)
    w5t = jnp.stack([Wl[0], Wl[1], Wr[0], Wr[1], We[0]], axis=1)
    att_col = att.reshape(HC, 1)

    co = _sc_gather(x.reshape(-1), src, dst)
    ex = _tc_dense(co, ea, w5t, att_col)
    t = _sc_scatter(ex, co, dst_r)
    st = _tc_stats(t)
    return _tc_out(t, st, Wl, gamma.reshape(1, HC), beta.reshape(1, HC))


# speedup vs baseline: 155.1797x; 1.6629x over previous
import functools

import jax
import jax.numpy as jnp
from jax import lax
from jax.experimental import pallas as pl
from jax.experimental.pallas import tpu as pltpu
from jax.experimental.pallas import tpu_sc as plsc

N = 50000
E = 800000
H = 4
C = 32
HC = 128

NC = 2
NS = 16
NW = NC * NS
W_PER = 25600
EP = NW * W_PER
CH = 1024
CHUNKS = W_PER // CH
GROUPS = CH // 16
NP = 50048
ZR = NP // NS

_mesh = plsc.VectorSubcoreMesh(
    core_axis_name="c", subcore_axis_name="s", num_cores=NC, num_subcores=NS)
_sc_params = pltpu.CompilerParams(
    needs_layout_passes=False, use_tc_tiling_on_sc=False)


@functools.partial(
    pl.kernel,
    out_type=jax.ShapeDtypeStruct((4, EP), jnp.float32),
    mesh=_mesh,
    scratch_types=[
        pltpu.VMEM((2 * N,), jnp.float32),
        pltpu.VMEM((CH,), jnp.int32),
        pltpu.VMEM((CH,), jnp.int32),
        pltpu.VMEM((4, CH), jnp.float32),
    ],
    compiler_params=_sc_params,
)
def _sc_gather(x_hbm, src_hbm, dst_hbm, out_hbm, xtab, sbuf, dbuf, cbuf):
    c = lax.axis_index("c")
    s = lax.axis_index("s")
    wid = s * NC + c
    pltpu.sync_copy(x_hbm, xtab)
    base = wid * W_PER

    @pl.loop(0, CHUNKS)
    def _chunk(ci):
        off = pl.multiple_of(base + ci * CH, CH)
        pltpu.sync_copy(src_hbm.at[pl.ds(off, CH)], sbuf)
        pltpu.sync_copy(dst_hbm.at[pl.ds(off, CH)], dbuf)

        @pl.loop(0, GROUPS)
        def _grp(g):
            si = sbuf[pl.ds(g * 16, 16)] * 2
            di = dbuf[pl.ds(g * 16, 16)] * 2
            cbuf[0, pl.ds(g * 16, 16)] = plsc.load_gather(xtab, [si])
            cbuf[1, pl.ds(g * 16, 16)] = plsc.load_gather(xtab, [si + 1])
            cbuf[2, pl.ds(g * 16, 16)] = plsc.load_gather(xtab, [di])
            cbuf[3, pl.ds(g * 16, 16)] = plsc.load_gather(xtab, [di + 1])

        pltpu.sync_copy(cbuf, out_hbm.at[:, pl.ds(off, CH)])


@functools.partial(
    pl.kernel,
    out_type=jax.ShapeDtypeStruct((NC, NP, 16), jnp.float32),
    mesh=_mesh,
    scratch_types=[
        pltpu.VMEM_SHARED((NP, 16), jnp.float32),
        pltpu.VMEM((8, 128), jnp.int32),
        pltpu.VMEM((4, CH), jnp.float32),
        pltpu.VMEM((2, CH), jnp.float32),
        pltpu.VMEM((CH, 16), jnp.float32),
        pltpu.VMEM((ZR, 16), jnp.float32),
    ],
    compiler_params=_sc_params,
)
def _sc_scatter(ex_hbm, co_hbm, dst_hbm, out_hbm,
                tsh, dbuf, exbuf, abbuf, vbuf, zbuf):
    c = lax.axis_index("c")
    s = lax.axis_index("s")
    wid = s * NC + c

    @pl.loop(0, ZR)
    def _z(i):
        zbuf[i, :] = jnp.zeros((16,), jnp.float32)

    @pl.loop(0, CH)
    def _zv(i):
        vbuf[i, :] = jnp.zeros((16,), jnp.float32)

    pltpu.sync_copy(zbuf, tsh.at[pl.ds(s * ZR, ZR)])
    plsc.subcore_barrier()

    base = wid * W_PER

    @pl.loop(0, CHUNKS)
    def _chunk(ci):
        off = pl.multiple_of(base + ci * CH, CH)
        pltpu.sync_copy(ex_hbm.at[:, pl.ds(off, CH)], exbuf)
        pltpu.sync_copy(co_hbm.at[pl.ds(0, 2), pl.ds(off, CH)], abbuf)
        pltpu.sync_copy(dst_hbm.at[pl.ds(pl.multiple_of(off // 128, 8), 8)],
                        dbuf)

        @pl.loop(0, GROUPS)
        def _grp(g):
            rows = g * 16 + lax.iota(jnp.int32, 16)
            a = abbuf[0, pl.ds(g * 16, 16)]
            b = abbuf[1, pl.ds(g * 16, 16)]
            for h in range(H):
                exh = exbuf[h, pl.ds(g * 16, 16)]
                col = jnp.full((16,), h, jnp.int32)
                plsc.store_scatter(vbuf, [rows, col], exh)
                plsc.store_scatter(vbuf, [rows, col + 4], exh * a)
                plsc.store_scatter(vbuf, [rows, col + 8], exh * b)

        for j in range(8):
            pltpu.sync_copy(vbuf.at[pl.ds(j * 128, 128)],
                            tsh.at[dbuf.at[j]], add=True)

    plsc.subcore_barrier()
    pltpu.sync_copy(tsh.at[pl.ds(s * ZR, ZR)],
                    out_hbm.at[c, pl.ds(s * ZR, ZR)])


EB = 4096


def _tc_dense_body(co_ref, ea_ref, w5t_ref, hsa_ref, out_ref):
    coef5 = jnp.concatenate([co_ref[...], ea_ref[...]], axis=0)
    w5t = w5t_ref[...]
    m = w5t[:, 0:1] * coef5[0:1, :]
    for k in range(1, 5):
        m = m + w5t[:, k:k + 1] * coef5[k:k + 1, :]
    g = jnp.maximum(m, 0.2 * m)
    logits = lax.dot_general(hsa_ref[...], g, (((1,), (0,)), ((), ())),
                             preferred_element_type=jnp.float32)
    eidx = pl.program_id(0) * EB + lax.broadcasted_iota(jnp.int32, (H, EB), 1)
    out_ref[...] = jnp.where(eidx < E, jnp.exp(logits), 0.0)


_tc_dense = pl.pallas_call(
    _tc_dense_body,
    grid=(EP // EB,),
    in_specs=[
        pl.BlockSpec((4, EB), lambda i: (0, i)),
        pl.BlockSpec((1, EB), lambda i: (0, i)),
        pl.BlockSpec((HC, 5), lambda i: (0, 0)),
        pl.BlockSpec((H, HC), lambda i: (0, 0)),
    ],
    out_specs=pl.BlockSpec((4, EB), lambda i: (0, i)),
    out_shape=jax.ShapeDtypeStruct((4, EP), jnp.float32),
)


NB = 2000


def _tc_stats_body(t_ref, wl_ref, gam_ref, bet_ref, acc_ref, g_ref):
    i = pl.program_id(0)
    ts = t_ref[0] + t_ref[1]
    den = ts[:, 0:4] + 1e-16
    sa = ts[:, 4:8] / den
    sb = ts[:, 8:12] / den
    m1 = jnp.sum(jnp.concatenate([sa, sb], axis=1), axis=0, keepdims=True)
    m2 = jnp.sum(jnp.concatenate([sa * sa, sb * sb], axis=1),
                 axis=0, keepdims=True)
    m3 = jnp.sum(jnp.concatenate([sa * sb, sa * sb], axis=1),
                 axis=0, keepdims=True)
    blk = jnp.concatenate([m1, m2, m3], axis=0)
    blk = jnp.concatenate([blk, jnp.zeros((3, 120), jnp.float32)], axis=1)
    blk = jnp.concatenate([blk, jnp.zeros((5, 128), jnp.float32)], axis=0)

    @pl.when(i == 0)
    def _():
        acc_ref[...] = jnp.zeros_like(acc_ref)

    acc_ref[...] += blk

    @pl.when(i == N // NB - 1)
    def _():
        acc = acc_ref[...]
        inv_n = 1.0 / N
        m1f = acc[0:1, 0:8] * inv_n
        m2f = acc[1:2, 0:8] * inv_n
        m3f = acc[2:3, 0:8] * inv_n
        am, bm = m1f[:, 0:4], m1f[:, 4:8]
        var_a = m2f[:, 0:4] - am * am
        var_b = m2f[:, 4:8] - bm * bm
        cov = m3f[:, 0:4] - am * bm
        hs = (lax.broadcasted_iota(jnp.int32, (H, HC), 1) // C
              == lax.broadcasted_iota(jnp.int32, (H, HC), 0)
              ).astype(jnp.float32)

        def expand(z):
            return lax.dot_general(z, hs, (((1,), (0,)), ((), ())),
                                   preferred_element_type=jnp.float32,
                                   precision=lax.Precision.HIGHEST)

        u = wl_ref[0:1, :]
        v = wl_ref[1:2, :]
        var = (u * u * expand(var_a) + v * v * expand(var_b)
               + 2.0 * u * v * expand(cov))
        sig = jnp.sqrt(var + 1e-5)
        p = gam_ref[...] * u / sig
        q = gam_ref[...] * v / sig
        const = bet_ref[...] - expand(am) * p - expand(bm) * q
        g_rows = jnp.concatenate(
            [hs * p, hs * q, const, jnp.zeros((7, HC), jnp.float32)], axis=0)
        g_ref[...] = g_rows


_tc_stats = pl.pallas_call(
    _tc_stats_body,
    grid=(N // NB,),
    in_specs=[
        pl.BlockSpec((NC, NB, 16), lambda i: (0, i, 0)),
        pl.BlockSpec((2, HC), lambda i: (0, 0)),
        pl.BlockSpec((1, HC), lambda i: (0, 0)),
        pl.BlockSpec((1, HC), lambda i: (0, 0)),
    ],
    out_specs=[pl.BlockSpec((8, 128), lambda i: (0, 0)),
               pl.BlockSpec((16, HC), lambda i: (0, 0))],
    out_shape=[jax.ShapeDtypeStruct((8, 128), jnp.float32),
               jax.ShapeDtypeStruct((16, HC), jnp.float32)],
)


def _tc_out_body(t_ref, g_ref, y_ref):
    ts = t_ref[0] + t_ref[1]
    den = ts[:, 0:4] + 1e-16
    sa = ts[:, 4:8] / den
    sb = ts[:, 8:12] / den
    z = jnp.concatenate(
        [sa, sb, jnp.ones((NB, 1), jnp.float32),
         jnp.zeros((NB, 7), jnp.float32)], axis=1)
    y_ref[...] = lax.dot_general(z, g_ref[...], (((1,), (0,)), ((), ())),
                                 preferred_element_type=jnp.float32,
                                 precision=lax.Precision.HIGHEST)


_tc_out = pl.pallas_call(
    _tc_out_body,
    grid=(N // NB,),
    in_specs=[
        pl.BlockSpec((NC, NB, 16), lambda i: (0, i, 0)),
        pl.BlockSpec((16, HC), lambda i: (0, 0)),
    ],
    out_specs=pl.BlockSpec((NB, HC), lambda i: (i, 0)),
    out_shape=jax.ShapeDtypeStruct((N, HC), jnp.float32),
)


def kernel(x, edge_index, edge_attr, Wl, Wr, We, att, bias, gamma, beta):
    del bias
    pad = EP - E
    src = jnp.concatenate([edge_index[0], jnp.zeros((pad,), jnp.int32)])
    dst = jnp.concatenate([edge_index[1], jnp.zeros((pad,), jnp.int32)])
    dst_r = dst.reshape(EP // 128, 128)
    ea = jnp.concatenate([edge_attr[:, 0],
                          jnp.zeros((pad,), jnp.float32)]).reshape(1, EP)
    w5t = jnp.stack([Wl[0], Wl[1], Wr[0], Wr[1], We[0]], axis=1)
    hmap = jnp.arange(HC, dtype=jnp.int32) // C
    hsa = jnp.where(hmap[None, :] == jnp.arange(H, dtype=jnp.int32)[:, None],
                    att.reshape(1, HC), 0.0)

    co = _sc_gather(x.reshape(-1), src, dst)
    ex = _tc_dense(co, ea, w5t, hsa)
    t = _sc_scatter(ex, co, dst_r)
    _, g = _tc_stats(t, Wl, gamma.reshape(1, HC), beta.reshape(1, HC))
    return _tc_out(t, g)
